# baseline (speedup 1.0000x reference)
import functools
import jax
import jax.numpy as jnp
from jax import lax
from jax.experimental import pallas as pl
from jax.experimental.pallas import tpu as pltpu
from jax.experimental.pallas import tpu_sc as plsc

N = 50000
E = 800000
IN_DIM = 12
HID = 64
HALF = 32
ODIM = 2
NLAYERS = 3

NP = 51200
NPQ = NP // 4

NC = 2
NS = 16
EPT = E // NS
CH = 200
NCHUNK = EPT // CH
RPT = 3128
LAST_BASE = N - RPT
NZ = RPT // CH
REM = RPT - NZ * CH
RPTP = NP // NS
NPAD_CH = (NP - N) // CH


def _agg_body(tab_lo, tab_hi, srcp, dstp, *refs, with_deg):
    if with_deg:
        (out_lo, out_hi, deg_out,
         src0, dst0, rows0, src1, dst1, rows1, src2, dst2, rows2,
         ones_v, acc, acc_deg,
         semI0, semG0, semS0, semI1, semG1, semS1, semI2, semG2, semS2) = refs
    else:
        (out_lo, out_hi,
         src0, dst0, rows0, src1, dst1, rows1, src2, dst2, rows2, acc,
         semI0, semG0, semS0, semI1, semG1, semS1, semI2, semG2, semS2) = refs
    c = lax.axis_index("c")
    s = lax.axis_index("s")
    zero16 = jnp.zeros((16,), jnp.float32)
    one16 = jnp.ones((16,), jnp.float32)

    def zrow(j, carry):
        rows0[j, pl.ds(0, 16)] = zero16
        rows0[j, pl.ds(16, 16)] = zero16
        return carry

    lax.fori_loop(0, CH, zrow, 0)

    base = pl.multiple_of(jnp.where(s == NS - 1, LAST_BASE, s * RPT), 8)

    def zcp(j, carry):
        pltpu.sync_copy(rows0, acc.at[pl.ds(base + j * CH, CH)])
        return carry

    lax.fori_loop(0, NZ, zcp, 0)
    pltpu.sync_copy(rows0.at[pl.ds(0, REM)], acc.at[pl.ds(base + NZ * CH, REM)])

    if with_deg:
        def fill0(j, carry):
            ones_v[pl.ds(j * 16, 16)] = zero16
            return carry

        lax.fori_loop(0, CH // 16, fill0, 0)
        ones_v[pl.ds(CH - 16, 16)] = zero16
        basep = pl.multiple_of(s * RPTP, 8)

        def zdeg(j, carry):
            pltpu.sync_copy(ones_v, acc_deg.at[pl.ds(basep + j * CH, CH)])
            return carry

        lax.fori_loop(0, RPTP // CH, zdeg, 0)

        def fill1(j, carry):
            ones_v[pl.ds(j * 16, 16)] = one16
            return carry

        lax.fori_loop(0, CH // 16, fill1, 0)
        ones_v[pl.ds(CH - 16, 16)] = one16

    plsc.subcore_barrier()

    ebase = s * EPT
    slots = ((src0, dst0, rows0, semI0, semG0, semS0),
             (src1, dst1, rows1, semI1, semG1, semS1),
             (src2, dst2, rows2, semI2, semG2, semS2))

    def issue_i(k, sl):
        off = pl.multiple_of(ebase + k * CH, 8)
        pltpu.async_copy(srcp.at[pl.ds(off, CH)], sl[0], sl[3])
        pltpu.async_copy(dstp.at[pl.ds(off, CH)], sl[1], sl[3])

    def wait_i(sl):
        pltpu.make_async_copy(srcp.at[pl.ds(0, CH)], sl[0], sl[3]).wait()
        pltpu.make_async_copy(dstp.at[pl.ds(0, CH)], sl[1], sl[3]).wait()

    def issue_g(sl):
        @pl.when(c == 0)
        def _():
            pltpu.async_copy(tab_lo.at[sl[0]], sl[2], sl[4])

        @pl.when(c == 1)
        def _():
            pltpu.async_copy(tab_hi.at[sl[0]], sl[2], sl[4])

    def wait_g(sl):
        pltpu.make_async_copy(tab_lo.at[sl[0]], sl[2], sl[4]).wait()

    def issue_s(sl):
        pltpu.async_copy(sl[2], acc.at[sl[1]], sl[5], add=True)
        if with_deg:
            pltpu.async_copy(ones_v, acc_deg.at[sl[1]], sl[5], add=True)

    def wait_s(sl):
        pltpu.make_async_copy(sl[2], acc.at[sl[1]], sl[5]).wait()
        if with_deg:
            pltpu.make_async_copy(ones_v, acc_deg.at[sl[1]], sl[5]).wait()

    def turn(k, b, first=False, last=False):
        sl = slots[b]
        nx = slots[(b + 1) % 3]
        pv = slots[(b + 2) % 3]
        if not last:
            wait_i(nx)
            issue_g(nx)
        wait_g(sl)
        issue_s(sl)
        if not first:
            wait_s(pv)
        if not last:
            @pl.when(k + 2 < NCHUNK)
            def _():
                issue_i(k + 2, pv)

    issue_i(0, slots[0])
    wait_i(slots[0])
    issue_g(slots[0])
    issue_i(1, slots[1])
    turn(0, 0, first=True)
    turn(1, 1)

    def triple(t, carry):
        for j, b in ((2, 2), (3, 0), (4, 1)):
            turn(3 * t + j, b)
        return carry

    lax.fori_loop(0, (NCHUNK - 4) // 3, triple, 0)

    turn(NCHUNK - 2, (NCHUNK - 2) % 3)
    turn(NCHUNK - 1, (NCHUNK - 1) % 3, last=True)

    wait_s(slots[(NCHUNK - 1) % 3])
    plsc.subcore_barrier()

    @pl.when(c == 0)
    def _():
        pltpu.sync_copy(acc.at[pl.ds(base, RPT)], out_lo.at[pl.ds(base, RPT)])

    @pl.when(c == 1)
    def _():
        pltpu.sync_copy(acc.at[pl.ds(base, RPT)], out_hi.at[pl.ds(base, RPT)])

    if with_deg:
        basep = pl.multiple_of(s * RPTP, 8)

        def wdeg(j, carry):
            pltpu.sync_copy(acc_deg.at[pl.ds(basep + j * CH, CH)], ones_v)
            pltpu.sync_copy(ones_v, deg_out.at[pl.ds(basep + j * CH, CH)])
            return carry

        lax.fori_loop(0, RPTP // CH, wdeg, 0)

    @pl.when(s == 0)
    def _():
        lax.fori_loop(0, CH, zrow, 0)

        def pz(j, carry):
            @pl.when(c == 0)
            def _():
                pltpu.sync_copy(rows0, out_lo.at[pl.ds(N + j * CH, CH)])

            @pl.when(c == 1)
            def _():
                pltpu.sync_copy(rows0, out_hi.at[pl.ds(N + j * CH, CH)])

            return carry

        lax.fori_loop(0, NPAD_CH, pz, 0)


@functools.cache
def _sc_kernels():
    mesh = plsc.VectorSubcoreMesh(core_axis_name="c", subcore_axis_name="s",
                                  num_cores=NC, num_subcores=NS)
    tab = jax.ShapeDtypeStruct((NP, HALF), jnp.float32)
    sems = [pltpu.SemaphoreType.DMA] * 9
    slot_bufs = [
        pltpu.VMEM((CH,), jnp.int32),
        pltpu.VMEM((CH,), jnp.int32),
        pltpu.VMEM((CH, HALF), jnp.float32),
    ] * 3
    agg0 = functools.partial(
        pl.kernel,
        out_type=[tab, tab, jax.ShapeDtypeStruct((NP,), jnp.float32)],
        mesh=mesh,
        compiler_params=pltpu.CompilerParams(use_tc_tiling_on_sc=False),
        scratch_types=slot_bufs + [
            pltpu.VMEM((CH,), jnp.float32),
            pltpu.VMEM_SHARED((N, HALF), jnp.float32),
            pltpu.VMEM_SHARED((NP,), jnp.float32),
        ] + sems,
    )(functools.partial(_agg_body, with_deg=True))
    agg = functools.partial(
        pl.kernel,
        out_type=[tab, tab],
        mesh=mesh,
        compiler_params=pltpu.CompilerParams(use_tc_tiling_on_sc=False),
        scratch_types=slot_bufs + [
            pltpu.VMEM_SHARED((N, HALF), jnp.float32),
        ] + sems,
    )(functools.partial(_agg_body, with_deg=False))
    return agg0, agg


BROW = 640
GRID = NPQ // BROW


def _embed_body(f_ref, p_ref, q_ref, blo_ref, bhi_ref, lo_ref, hi_ref):
    f = f_ref[...]
    lo_ref[...] = jnp.dot(f, p_ref[...], preferred_element_type=jnp.float32) + blo_ref[...]
    hi_ref[...] = jnp.dot(f, q_ref[...], preferred_element_type=jnp.float32) + bhi_ref[...]


def _layer_body(tl_ref, th_ref, nl_ref, nh_ref, dg_ref,
                sa, sb, sc_, sd, na, nb, ncc, nd, blo_ref, bhi_ref,
                lo_ref, hi_ref, *, residual):
    tl = tl_ref[...]
    th = th_ref[...]
    nl = nl_ref[...]
    nh = nh_ref[...]
    invd = 1.0 / jnp.maximum(dg_ref[...], 1.0)
    dot = functools.partial(jnp.dot, preferred_element_type=jnp.float32)
    xlo = dot(tl, sa[...]) + dot(th, sb[...]) + (dot(nl, na[...]) + dot(nh, nb[...])) * invd + blo_ref[...]
    xhi = dot(tl, sc_[...]) + dot(th, sd[...]) + (dot(nl, ncc[...]) + dot(nh, nd[...])) * invd + bhi_ref[...]
    xlo = jnp.maximum(xlo, 0.0)
    xhi = jnp.maximum(xhi, 0.0)
    if residual:
        xlo = xlo + tl
        xhi = xhi + th
    lo_ref[...] = xlo
    hi_ref[...] = xhi


def _layer_head_body(tl_ref, th_ref, nl_ref, nh_ref, dg_ref,
                     sa, sb, sc_, sd, na, nb, ncc, nd, blo_ref, bhi_ref,
                     w1a, w1b, b1_ref, w2_ref, b2_ref, out_ref):
    tl = tl_ref[...]
    th = th_ref[...]
    nl = nl_ref[...]
    nh = nh_ref[...]
    invd = 1.0 / jnp.maximum(dg_ref[...], 1.0)
    dot = functools.partial(jnp.dot, preferred_element_type=jnp.float32)
    xlo = dot(tl, sa[...]) + dot(th, sb[...]) + (dot(nl, na[...]) + dot(nh, nb[...])) * invd + blo_ref[...]
    xhi = dot(tl, sc_[...]) + dot(th, sd[...]) + (dot(nl, ncc[...]) + dot(nh, nd[...])) * invd + bhi_ref[...]
    xlo = jnp.maximum(xlo, 0.0) + tl
    xhi = jnp.maximum(xhi, 0.0) + th
    hid = dot(xlo, w1a[...]) + dot(xhi, w1b[...]) + b1_ref[...]
    hid = jnp.maximum(hid, 0.0)
    out_ref[...] = dot(hid, w2_ref[...]) + b2_ref[...]


def _blk(minor):
    return pl.BlockSpec((BROW, minor), lambda i: (i, 0))


def _full(shape):
    return pl.BlockSpec(shape, lambda i: tuple(0 for _ in shape))


_PACKED = jax.ShapeDtypeStruct((NPQ, 128), jnp.float32)


def _embed_call(fpack, pbd, qbd, blo, bhi):
    return pl.pallas_call(
        _embed_body,
        grid=(GRID,),
        in_specs=[_blk(4 * IN_DIM), _full((4 * IN_DIM, 128)), _full((4 * IN_DIM, 128)),
                  _full((1, 128)), _full((1, 128))],
        out_specs=[_blk(128), _blk(128)],
        out_shape=[_PACKED, _PACKED],
    )(fpack, pbd, qbd, blo, bhi)


def _layer_call(residual, tl, th, nl, nh, dg, ws, blo, bhi):
    return pl.pallas_call(
        functools.partial(_layer_body, residual=residual),
        grid=(GRID,),
        in_specs=[_blk(128)] * 5 + [_full((128, 128))] * 8 + [_full((1, 128))] * 2,
        out_specs=[_blk(128), _blk(128)],
        out_shape=[_PACKED, _PACKED],
    )(tl, th, nl, nh, dg, *ws, blo, bhi)


def _layer_head_call(tl, th, nl, nh, dg, ws, blo, bhi, w1a, w1b, b1p, w2bd, b2p):
    return pl.pallas_call(
        _layer_head_body,
        grid=(GRID,),
        in_specs=[_blk(128)] * 5 + [_full((128, 128))] * 8 + [_full((1, 128))] * 2
                 + [_full((128, 128)), _full((128, 128)), _full((1, 128)),
                    _full((128, 4 * ODIM)), _full((1, 4 * ODIM))],
        out_specs=_blk(4 * ODIM),
        out_shape=jax.ShapeDtypeStruct((NPQ, 4 * ODIM), jnp.float32),
    )(tl, th, nl, nh, dg, *ws, blo, bhi, w1a, w1b, b1p, w2bd, b2p)


def kernel(features, edge_index, W_emb, b_emb, W_self, W_neigh, b_sage,
           bn_gamma, bn_beta, bn_mean, bn_var, W1, b1, W2, b2):
    ei = edge_index.astype(jnp.int32)
    srcp = ei[0]
    dstp = ei[1]

    scale = bn_gamma * lax.rsqrt(bn_var + 1e-5)
    bf = (b_sage - bn_mean) * scale + bn_beta
    Wsf = W_self * scale[:, :, None]
    Wnf = W_neigh * scale[:, :, None]
    eye4 = jnp.eye(4, dtype=jnp.float32)
    bd = lambda m: jnp.kron(eye4, m)
    layer_ws = []
    layer_bs = []
    for i in range(NLAYERS):
        ws = [bd(Wsf[i, :HALF, :HALF].T), bd(Wsf[i, :HALF, HALF:].T),
              bd(Wsf[i, HALF:, :HALF].T), bd(Wsf[i, HALF:, HALF:].T),
              bd(Wnf[i, :HALF, :HALF].T), bd(Wnf[i, :HALF, HALF:].T),
              bd(Wnf[i, HALF:, :HALF].T), bd(Wnf[i, HALF:, HALF:].T)]
        layer_ws.append(ws)
        layer_bs.append((jnp.tile(bf[i, :HALF], 4)[None, :],
                         jnp.tile(bf[i, HALF:], 4)[None, :]))
    pbd = bd(W_emb[:HALF, :].T)
    qbd = bd(W_emb[HALF:, :].T)
    eblo = jnp.tile(b_emb[:HALF], 4)[None, :]
    ebhi = jnp.tile(b_emb[HALF:], 4)[None, :]
    w1a = bd(W1[:, :HALF].T)
    w1b = bd(W1[:, HALF:].T)
    b1p = jnp.tile(b1, 4)[None, :]
    w2bd = bd(W2.T)
    b2p = jnp.tile(b2, 4)[None, :]

    fpack = jnp.pad(features.reshape(N // 4, 4 * IN_DIM), ((0, NPQ - N // 4), (0, 0)))

    agg0k, aggk = _sc_kernels()
    hl, hh = _embed_call(fpack, pbd, qbd, eblo, ebhi)
    degrep = None
    for i in range(NLAYERS):
        if i == 0:
            nl, nh, deg = agg0k(hl.reshape(NP, HALF), hh.reshape(NP, HALF),
                                srcp, dstp)
            degrep = jnp.repeat(deg, HALF).reshape(NPQ, 128)
        else:
            nl, nh = aggk(hl.reshape(NP, HALF), hh.reshape(NP, HALF),
                          srcp, dstp)
        nlp = nl.reshape(NPQ, 128)
        nhp = nh.reshape(NPQ, 128)
        if i < NLAYERS - 1:
            hl, hh = _layer_call(i > 0, hl, hh, nlp, nhp,
                                 degrep, layer_ws[i], *layer_bs[i])
        else:
            out = _layer_head_call(hl, hh, nlp, nhp, degrep,
                                   layer_ws[i], *layer_bs[i],
                                   w1a=w1a, w1b=w1b, b1p=b1p, w2bd=w2bd, b2p=b2p)
    return out.reshape(NP, ODIM)[:N]


# speedup vs baseline: 1.1136x; 1.1136x over previous
import functools
import jax
import jax.numpy as jnp
from jax import lax
from jax.experimental import pallas as pl
from jax.experimental.pallas import tpu as pltpu
from jax.experimental.pallas import tpu_sc as plsc

N = 50000
E = 800000
IN_DIM = 12
HID = 64
HALF = 32
ODIM = 2
NLAYERS = 3

NP = 51200
NPQ = NP // 4

NC = 2
NS = 16
EPT = E // NS
CH = 400
NCHUNK = EPT // CH
RPT = 3128
LAST_BASE = N - RPT
NZ = RPT // CH
REM = RPT - NZ * CH
RPTP = NP // NS
NPAD_CH = (NP - N) // CH


def _agg_body(tab_lo, tab_hi, srcp, dstp, *refs, with_deg):
    if with_deg:
        (out_lo, out_hi, deg_out,
         src0, dst0, rows0, src1, dst1, rows1, ones_v, acc, acc_deg,
         semI0, semG0, semS0, semI1, semG1, semS1) = refs
    else:
        (out_lo, out_hi,
         src0, dst0, rows0, src1, dst1, rows1, acc,
         semI0, semG0, semS0, semI1, semG1, semS1) = refs
    c = lax.axis_index("c")
    s = lax.axis_index("s")
    zero16 = jnp.zeros((16,), jnp.float32)
    one16 = jnp.ones((16,), jnp.float32)

    def zrow(j, carry):
        rows0[j, pl.ds(0, 16)] = zero16
        rows0[j, pl.ds(16, 16)] = zero16
        return carry

    lax.fori_loop(0, CH, zrow, 0)

    base = pl.multiple_of(jnp.where(s == NS - 1, LAST_BASE, s * RPT), 8)

    def zcp(j, carry):
        pltpu.sync_copy(rows0, acc.at[pl.ds(base + j * CH, CH)])
        return carry

    lax.fori_loop(0, NZ, zcp, 0)
    pltpu.sync_copy(rows0.at[pl.ds(0, REM)], acc.at[pl.ds(base + NZ * CH, REM)])

    if with_deg:
        def fill0(j, carry):
            ones_v[pl.ds(j * 16, 16)] = zero16
            return carry

        lax.fori_loop(0, CH // 16, fill0, 0)
        basep = pl.multiple_of(s * RPTP, 8)

        def zdeg(j, carry):
            pltpu.sync_copy(ones_v, acc_deg.at[pl.ds(basep + j * CH, CH)])
            return carry

        lax.fori_loop(0, RPTP // CH, zdeg, 0)

        def fill1(j, carry):
            ones_v[pl.ds(j * 16, 16)] = one16
            return carry

        lax.fori_loop(0, CH // 16, fill1, 0)

    plsc.subcore_barrier()

    ebase = s * EPT
    slots = ((src0, dst0, rows0, semI0, semG0, semS0),
             (src1, dst1, rows1, semI1, semG1, semS1))

    def issue_i(k, sl):
        off = pl.multiple_of(ebase + k * CH, 8)
        pltpu.async_copy(srcp.at[pl.ds(off, CH)], sl[0], sl[3])
        pltpu.async_copy(dstp.at[pl.ds(off, CH)], sl[1], sl[3])

    def wait_i(sl):
        pltpu.make_async_copy(srcp.at[pl.ds(0, CH)], sl[0], sl[3]).wait()
        pltpu.make_async_copy(dstp.at[pl.ds(0, CH)], sl[1], sl[3]).wait()

    def issue_g(sl):
        @pl.when(c == 0)
        def _():
            pltpu.async_copy(tab_lo.at[sl[0]], sl[2], sl[4])

        @pl.when(c == 1)
        def _():
            pltpu.async_copy(tab_hi.at[sl[0]], sl[2], sl[4])

    def wait_g(sl):
        pltpu.make_async_copy(tab_lo.at[sl[0]], sl[2], sl[4]).wait()

    def issue_s(sl):
        pltpu.async_copy(sl[2], acc.at[sl[1]], sl[5], add=True)
        if with_deg:
            pltpu.async_copy(ones_v, acc_deg.at[sl[1]], sl[5], add=True)

    def wait_s(sl):
        pltpu.make_async_copy(sl[2], acc.at[sl[1]], sl[5]).wait()
        if with_deg:
            pltpu.make_async_copy(ones_v, acc_deg.at[sl[1]], sl[5]).wait()

    issue_i(0, slots[0])
    wait_i(slots[0])
    issue_g(slots[0])
    issue_i(1, slots[1])
    wait_g(slots[0])
    issue_s(slots[0])

    def pair(p, carry):
        for b in (1, 0):
            k = 2 * p + (1 if b == 1 else 2)
            sl = slots[b]
            ot = slots[1 - b]
            wait_i(sl)
            issue_g(sl)
            wait_s(ot)

            @pl.when(k + 1 < NCHUNK)
            def _():
                issue_i(k + 1, ot)

            wait_g(sl)
            issue_s(sl)
        return carry

    lax.fori_loop(0, (NCHUNK - 1) // 2, pair, 0)

    wait_s(slots[0])
    plsc.subcore_barrier()

    @pl.when(c == 0)
    def _():
        pltpu.sync_copy(acc.at[pl.ds(base, RPT)], out_lo.at[pl.ds(base, RPT)])

    @pl.when(c == 1)
    def _():
        pltpu.sync_copy(acc.at[pl.ds(base, RPT)], out_hi.at[pl.ds(base, RPT)])

    if with_deg:
        basep = pl.multiple_of(s * RPTP, 8)

        def wdeg(j, carry):
            pltpu.sync_copy(acc_deg.at[pl.ds(basep + j * CH, CH)], ones_v)
            pltpu.sync_copy(ones_v, deg_out.at[pl.ds(basep + j * CH, CH)])
            return carry

        lax.fori_loop(0, RPTP // CH, wdeg, 0)

    @pl.when(s == 0)
    def _():
        lax.fori_loop(0, CH, zrow, 0)

        def pz(j, carry):
            @pl.when(c == 0)
            def _():
                pltpu.sync_copy(rows0, out_lo.at[pl.ds(N + j * CH, CH)])

            @pl.when(c == 1)
            def _():
                pltpu.sync_copy(rows0, out_hi.at[pl.ds(N + j * CH, CH)])

            return carry

        lax.fori_loop(0, NPAD_CH, pz, 0)


@functools.cache
def _sc_kernels():
    mesh = plsc.VectorSubcoreMesh(core_axis_name="c", subcore_axis_name="s",
                                  num_cores=NC, num_subcores=NS)
    tab = jax.ShapeDtypeStruct((NP, HALF), jnp.float32)
    sems = [pltpu.SemaphoreType.DMA] * 6
    slot_bufs = [
        pltpu.VMEM((CH,), jnp.int32),
        pltpu.VMEM((CH,), jnp.int32),
        pltpu.VMEM((CH, HALF), jnp.float32),
        pltpu.VMEM((CH,), jnp.int32),
        pltpu.VMEM((CH,), jnp.int32),
        pltpu.VMEM((CH, HALF), jnp.float32),
    ]
    agg0 = functools.partial(
        pl.kernel,
        out_type=[tab, tab, jax.ShapeDtypeStruct((NP,), jnp.float32)],
        mesh=mesh,
        compiler_params=pltpu.CompilerParams(use_tc_tiling_on_sc=False),
        scratch_types=slot_bufs + [
            pltpu.VMEM((CH,), jnp.float32),
            pltpu.VMEM_SHARED((N, HALF), jnp.float32),
            pltpu.VMEM_SHARED((NP,), jnp.float32),
        ] + sems,
    )(functools.partial(_agg_body, with_deg=True))
    agg = functools.partial(
        pl.kernel,
        out_type=[tab, tab],
        mesh=mesh,
        compiler_params=pltpu.CompilerParams(use_tc_tiling_on_sc=False),
        scratch_types=slot_bufs + [
            pltpu.VMEM_SHARED((N, HALF), jnp.float32),
        ] + sems,
    )(functools.partial(_agg_body, with_deg=False))
    return agg0, agg


BROW = 640
GRID = NPQ // BROW


def _embed_body(f_ref, p_ref, q_ref, blo_ref, bhi_ref, lo_ref, hi_ref):
    f = f_ref[...]
    lo_ref[...] = jnp.dot(f, p_ref[...], preferred_element_type=jnp.float32) + blo_ref[...]
    hi_ref[...] = jnp.dot(f, q_ref[...], preferred_element_type=jnp.float32) + bhi_ref[...]


def _self_body(tl_ref, th_ref, sa, sb, sc_, sd, blo_ref, bhi_ref, lo_ref, hi_ref):
    tl = tl_ref[...]
    th = th_ref[...]
    dot = functools.partial(jnp.dot, preferred_element_type=jnp.float32)
    lo_ref[...] = dot(tl, sa[...]) + dot(th, sb[...]) + blo_ref[...]
    hi_ref[...] = dot(tl, sc_[...]) + dot(th, sd[...]) + bhi_ref[...]


def _layer_body(slo_ref, shi_ref, tl_ref, th_ref, nl_ref, nh_ref, dg_ref,
                na, nb, ncc, nd, lo_ref, hi_ref, *, residual):
    nl = nl_ref[...]
    nh = nh_ref[...]
    invd = 1.0 / jnp.maximum(dg_ref[...], 1.0)
    dot = functools.partial(jnp.dot, preferred_element_type=jnp.float32)
    xlo = slo_ref[...] + (dot(nl, na[...]) + dot(nh, nb[...])) * invd
    xhi = shi_ref[...] + (dot(nl, ncc[...]) + dot(nh, nd[...])) * invd
    xlo = jnp.maximum(xlo, 0.0)
    xhi = jnp.maximum(xhi, 0.0)
    if residual:
        xlo = xlo + tl_ref[...]
        xhi = xhi + th_ref[...]
    lo_ref[...] = xlo
    hi_ref[...] = xhi


def _layer_head_body(slo_ref, shi_ref, tl_ref, th_ref, nl_ref, nh_ref, dg_ref,
                     na, nb, ncc, nd,
                     w1a, w1b, b1_ref, w2_ref, b2_ref, out_ref):
    nl = nl_ref[...]
    nh = nh_ref[...]
    invd = 1.0 / jnp.maximum(dg_ref[...], 1.0)
    dot = functools.partial(jnp.dot, preferred_element_type=jnp.float32)
    xlo = slo_ref[...] + (dot(nl, na[...]) + dot(nh, nb[...])) * invd
    xhi = shi_ref[...] + (dot(nl, ncc[...]) + dot(nh, nd[...])) * invd
    xlo = jnp.maximum(xlo, 0.0) + tl_ref[...]
    xhi = jnp.maximum(xhi, 0.0) + th_ref[...]
    hid = dot(xlo, w1a[...]) + dot(xhi, w1b[...]) + b1_ref[...]
    hid = jnp.maximum(hid, 0.0)
    out_ref[...] = dot(hid, w2_ref[...]) + b2_ref[...]


def _blk(minor):
    return pl.BlockSpec((BROW, minor), lambda i: (i, 0))


def _full(shape):
    return pl.BlockSpec(shape, lambda i: tuple(0 for _ in shape))


_PACKED = jax.ShapeDtypeStruct((NPQ, 128), jnp.float32)


def _embed_call(fpack, pbd, qbd, blo, bhi):
    return pl.pallas_call(
        _embed_body,
        grid=(GRID,),
        in_specs=[_blk(4 * IN_DIM), _full((4 * IN_DIM, 128)), _full((4 * IN_DIM, 128)),
                  _full((1, 128)), _full((1, 128))],
        out_specs=[_blk(128), _blk(128)],
        out_shape=[_PACKED, _PACKED],
    )(fpack, pbd, qbd, blo, bhi)


def _self_call(tl, th, ws, blo, bhi):
    return pl.pallas_call(
        _self_body,
        grid=(GRID,),
        in_specs=[_blk(128)] * 2 + [_full((128, 128))] * 4 + [_full((1, 128))] * 2,
        out_specs=[_blk(128), _blk(128)],
        out_shape=[_PACKED, _PACKED],
    )(tl, th, *ws[:4], blo, bhi)


def _layer_call(residual, slo, shi, tl, th, nl, nh, dg, ws):
    return pl.pallas_call(
        functools.partial(_layer_body, residual=residual),
        grid=(GRID,),
        in_specs=[_blk(128)] * 7 + [_full((128, 128))] * 4,
        out_specs=[_blk(128), _blk(128)],
        out_shape=[_PACKED, _PACKED],
    )(slo, shi, tl, th, nl, nh, dg, *ws[4:])


def _layer_head_call(slo, shi, tl, th, nl, nh, dg, ws, w1a, w1b, b1p, w2bd, b2p):
    return pl.pallas_call(
        _layer_head_body,
        grid=(GRID,),
        in_specs=[_blk(128)] * 7 + [_full((128, 128))] * 4
                 + [_full((128, 128)), _full((128, 128)), _full((1, 128)),
                    _full((128, 4 * ODIM)), _full((1, 4 * ODIM))],
        out_specs=_blk(4 * ODIM),
        out_shape=jax.ShapeDtypeStruct((NPQ, 4 * ODIM), jnp.float32),
    )(slo, shi, tl, th, nl, nh, dg, *ws[4:], w1a, w1b, b1p, w2bd, b2p)


def kernel(features, edge_index, W_emb, b_emb, W_self, W_neigh, b_sage,
           bn_gamma, bn_beta, bn_mean, bn_var, W1, b1, W2, b2):
    ei = edge_index.astype(jnp.int32)
    srcp = ei[0]
    dstp = ei[1]

    scale = bn_gamma * lax.rsqrt(bn_var + 1e-5)
    bf = (b_sage - bn_mean) * scale + bn_beta
    Wsf = W_self * scale[:, :, None]
    Wnf = W_neigh * scale[:, :, None]
    eye4 = jnp.eye(4, dtype=jnp.float32)
    bd = lambda m: jnp.kron(eye4, m)
    layer_ws = []
    layer_bs = []
    for i in range(NLAYERS):
        ws = [bd(Wsf[i, :HALF, :HALF].T), bd(Wsf[i, :HALF, HALF:].T),
              bd(Wsf[i, HALF:, :HALF].T), bd(Wsf[i, HALF:, HALF:].T),
              bd(Wnf[i, :HALF, :HALF].T), bd(Wnf[i, :HALF, HALF:].T),
              bd(Wnf[i, HALF:, :HALF].T), bd(Wnf[i, HALF:, HALF:].T)]
        layer_ws.append(ws)
        layer_bs.append((jnp.tile(bf[i, :HALF], 4)[None, :],
                         jnp.tile(bf[i, HALF:], 4)[None, :]))
    pbd = bd(W_emb[:HALF, :].T)
    qbd = bd(W_emb[HALF:, :].T)
    eblo = jnp.tile(b_emb[:HALF], 4)[None, :]
    ebhi = jnp.tile(b_emb[HALF:], 4)[None, :]
    w1a = bd(W1[:, :HALF].T)
    w1b = bd(W1[:, HALF:].T)
    b1p = jnp.tile(b1, 4)[None, :]
    w2bd = bd(W2.T)
    b2p = jnp.tile(b2, 4)[None, :]

    fpack = jnp.pad(features.reshape(N // 4, 4 * IN_DIM), ((0, NPQ - N // 4), (0, 0)))

    agg0k, aggk = _sc_kernels()
    hl, hh = _embed_call(fpack, pbd, qbd, eblo, ebhi)
    degrep = None
    for i in range(NLAYERS):
        if i == 0:
            nl, nh, deg = agg0k(hl.reshape(NP, HALF), hh.reshape(NP, HALF),
                                srcp, dstp)
            degrep = jnp.repeat(deg, HALF).reshape(NPQ, 128)
        else:
            nl, nh = aggk(hl.reshape(NP, HALF), hh.reshape(NP, HALF),
                          srcp, dstp)
        slo, shi = _self_call(hl, hh, layer_ws[i], *layer_bs[i])
        nlp = nl.reshape(NPQ, 128)
        nhp = nh.reshape(NPQ, 128)
        if i < NLAYERS - 1:
            hl, hh = _layer_call(i > 0, slo, shi, hl, hh, nlp, nhp,
                                 degrep, layer_ws[i])
        else:
            out = _layer_head_call(slo, shi, hl, hh, nlp, nhp, degrep,
                                   layer_ws[i],
                                   w1a=w1a, w1b=w1b, b1p=b1p, w2bd=w2bd, b2p=b2p)
    return out.reshape(NP, ODIM)[:N]


# speedup vs baseline: 1.1300x; 1.0148x over previous
import functools
import jax
import jax.numpy as jnp
from jax import lax
from jax.experimental import pallas as pl
from jax.experimental.pallas import tpu as pltpu
from jax.experimental.pallas import tpu_sc as plsc

N = 50000
E = 800000
IN_DIM = 12
HID = 64
HALF = 32
ODIM = 2
NLAYERS = 3

NP = 51200
NPQ = NP // 4

NC = 2
NS = 16
EPT = E // NS
CH = 400
NCHUNK = EPT // CH
RPT = 3128
LAST_BASE = N - RPT
NZ = RPT // CH
REM = RPT - NZ * CH
RPTP = NP // NS
NPAD_CH = (NP - N) // CH


def _agg_body(tab_lo, tab_hi, srcp, dstp, *refs, with_deg):
    if with_deg:
        (out_lo, out_hi, deg_out,
         src0, dst0, rows0, src1, dst1, rows1, ones_v, acc, acc_deg,
         semI0, semG0, semS0, semI1, semG1, semS1) = refs
    else:
        (out_lo, out_hi,
         src0, dst0, rows0, src1, dst1, rows1, acc,
         semI0, semG0, semS0, semI1, semG1, semS1) = refs
    c = lax.axis_index("c")
    s = lax.axis_index("s")
    zero16 = jnp.zeros((16,), jnp.float32)
    one16 = jnp.ones((16,), jnp.float32)

    def zrow(j, carry):
        rows0[j, pl.ds(0, 16)] = zero16
        rows0[j, pl.ds(16, 16)] = zero16
        return carry

    lax.fori_loop(0, CH, zrow, 0)

    base = pl.multiple_of(jnp.where(s == NS - 1, LAST_BASE, s * RPT), 8)

    def zcp(j, carry):
        pltpu.sync_copy(rows0, acc.at[pl.ds(base + j * CH, CH)])
        return carry

    lax.fori_loop(0, NZ, zcp, 0)
    pltpu.sync_copy(rows0.at[pl.ds(0, REM)], acc.at[pl.ds(base + NZ * CH, REM)])

    if with_deg:
        def fill0(j, carry):
            ones_v[pl.ds(j * 16, 16)] = zero16
            return carry

        lax.fori_loop(0, CH // 16, fill0, 0)
        basep = pl.multiple_of(s * RPTP, 8)

        def zdeg(j, carry):
            pltpu.sync_copy(ones_v, acc_deg.at[pl.ds(basep + j * CH, CH)])
            return carry

        lax.fori_loop(0, RPTP // CH, zdeg, 0)

        def fill1(j, carry):
            ones_v[pl.ds(j * 16, 16)] = one16
            return carry

        lax.fori_loop(0, CH // 16, fill1, 0)

    plsc.subcore_barrier()

    ebase = s * EPT
    slots = ((src0, dst0, rows0, semI0, semG0, semS0),
             (src1, dst1, rows1, semI1, semG1, semS1))

    def issue_i(k, sl):
        off = pl.multiple_of(ebase + k * CH, 8)
        pltpu.async_copy(srcp.at[pl.ds(off, CH)], sl[0], sl[3])
        pltpu.async_copy(dstp.at[pl.ds(off, CH)], sl[1], sl[3])

    def wait_i(sl):
        pltpu.make_async_copy(srcp.at[pl.ds(0, CH)], sl[0], sl[3]).wait()
        pltpu.make_async_copy(dstp.at[pl.ds(0, CH)], sl[1], sl[3]).wait()

    def issue_g(sl):
        @pl.when(c == 0)
        def _():
            pltpu.async_copy(tab_lo.at[sl[0]], sl[2], sl[4])

        @pl.when(c == 1)
        def _():
            pltpu.async_copy(tab_hi.at[sl[0]], sl[2], sl[4])

    def wait_g(sl):
        pltpu.make_async_copy(tab_lo.at[sl[0]], sl[2], sl[4]).wait()

    def issue_s(sl):
        pltpu.async_copy(sl[2], acc.at[sl[1]], sl[5], add=True)
        if with_deg:
            pltpu.async_copy(ones_v, acc_deg.at[sl[1]], sl[5], add=True)

    def wait_s(sl):
        pltpu.make_async_copy(sl[2], acc.at[sl[1]], sl[5]).wait()
        if with_deg:
            pltpu.make_async_copy(ones_v, acc_deg.at[sl[1]], sl[5]).wait()

    issue_i(0, slots[0])
    wait_i(slots[0])
    issue_g(slots[0])
    issue_i(1, slots[1])
    wait_g(slots[0])
    issue_s(slots[0])

    def pair(p, carry):
        for b in (1, 0):
            k = 2 * p + (1 if b == 1 else 2)
            sl = slots[b]
            ot = slots[1 - b]
            wait_i(sl)
            issue_g(sl)
            wait_s(ot)

            @pl.when(k + 1 < NCHUNK)
            def _():
                issue_i(k + 1, ot)

            wait_g(sl)
            issue_s(sl)
        return carry

    lax.fori_loop(0, (NCHUNK - 1) // 2, pair, 0)

    wait_s(slots[0])
    plsc.subcore_barrier()

    @pl.when(c == 0)
    def _():
        pltpu.sync_copy(acc.at[pl.ds(base, RPT)], out_lo.at[pl.ds(base, RPT)])

    @pl.when(c == 1)
    def _():
        pltpu.sync_copy(acc.at[pl.ds(base, RPT)], out_hi.at[pl.ds(base, RPT)])

    if with_deg:
        basep = pl.multiple_of(s * RPTP, 8)

        def wdeg(j, carry):
            pltpu.sync_copy(acc_deg.at[pl.ds(basep + j * CH, CH)], ones_v)
            pltpu.sync_copy(ones_v, deg_out.at[pl.ds(basep + j * CH, CH)])
            return carry

        lax.fori_loop(0, RPTP // CH, wdeg, 0)

    @pl.when(s == 0)
    def _():
        lax.fori_loop(0, CH, zrow, 0)

        def pz(j, carry):
            @pl.when(c == 0)
            def _():
                pltpu.sync_copy(rows0, out_lo.at[pl.ds(N + j * CH, CH)])

            @pl.when(c == 1)
            def _():
                pltpu.sync_copy(rows0, out_hi.at[pl.ds(N + j * CH, CH)])

            return carry

        lax.fori_loop(0, NPAD_CH, pz, 0)


@functools.cache
def _sc_kernels():
    mesh = plsc.VectorSubcoreMesh(core_axis_name="c", subcore_axis_name="s",
                                  num_cores=NC, num_subcores=NS)
    tab = jax.ShapeDtypeStruct((NP, HALF), jnp.float32)
    sems = [pltpu.SemaphoreType.DMA] * 6
    slot_bufs = [
        pltpu.VMEM((CH,), jnp.int32),
        pltpu.VMEM((CH,), jnp.int32),
        pltpu.VMEM((CH, HALF), jnp.float32),
        pltpu.VMEM((CH,), jnp.int32),
        pltpu.VMEM((CH,), jnp.int32),
        pltpu.VMEM((CH, HALF), jnp.float32),
    ]
    agg0 = functools.partial(
        pl.kernel,
        out_type=[tab, tab, jax.ShapeDtypeStruct((NP,), jnp.float32)],
        mesh=mesh,
        compiler_params=pltpu.CompilerParams(use_tc_tiling_on_sc=False),
        scratch_types=slot_bufs + [
            pltpu.VMEM((CH,), jnp.float32),
            pltpu.VMEM_SHARED((N, HALF), jnp.float32),
            pltpu.VMEM_SHARED((NP,), jnp.float32),
        ] + sems,
    )(functools.partial(_agg_body, with_deg=True))
    agg = functools.partial(
        pl.kernel,
        out_type=[tab, tab],
        mesh=mesh,
        compiler_params=pltpu.CompilerParams(use_tc_tiling_on_sc=False),
        scratch_types=slot_bufs + [
            pltpu.VMEM_SHARED((N, HALF), jnp.float32),
        ] + sems,
    )(functools.partial(_agg_body, with_deg=False))
    return agg0, agg


BROW = 640
GRID = NPQ // BROW


def _embed_body(f_ref, p_ref, q_ref, blo_ref, bhi_ref, lo_ref, hi_ref):
    f = f_ref[...]
    lo_ref[...] = jnp.dot(f, p_ref[...], preferred_element_type=jnp.float32) + blo_ref[...]
    hi_ref[...] = jnp.dot(f, q_ref[...], preferred_element_type=jnp.float32) + bhi_ref[...]


def _layer_body(tl_ref, th_ref, nl_ref, nh_ref, dg_ref,
                sa, sb, sc_, sd, na, nb, ncc, nd, blo_ref, bhi_ref,
                lo_ref, hi_ref, *, residual):
    tl = tl_ref[...]
    th = th_ref[...]
    nl = nl_ref[...]
    nh = nh_ref[...]
    invd = 1.0 / jnp.maximum(dg_ref[...], 1.0)
    dot = functools.partial(jnp.dot, preferred_element_type=jnp.float32)
    xlo = dot(tl, sa[...]) + dot(th, sb[...]) + (dot(nl, na[...]) + dot(nh, nb[...])) * invd + blo_ref[...]
    xhi = dot(tl, sc_[...]) + dot(th, sd[...]) + (dot(nl, ncc[...]) + dot(nh, nd[...])) * invd + bhi_ref[...]
    xlo = jnp.maximum(xlo, 0.0)
    xhi = jnp.maximum(xhi, 0.0)
    if residual:
        xlo = xlo + tl
        xhi = xhi + th
    lo_ref[...] = xlo
    hi_ref[...] = xhi


def _layer_head_body(tl_ref, th_ref, nl_ref, nh_ref, dg_ref,
                     sa, sb, sc_, sd, na, nb, ncc, nd, blo_ref, bhi_ref,
                     w1a, w1b, b1_ref, w2_ref, b2_ref, out_ref):
    tl = tl_ref[...]
    th = th_ref[...]
    nl = nl_ref[...]
    nh = nh_ref[...]
    invd = 1.0 / jnp.maximum(dg_ref[...], 1.0)
    dot = functools.partial(jnp.dot, preferred_element_type=jnp.float32)
    xlo = dot(tl, sa[...]) + dot(th, sb[...]) + (dot(nl, na[...]) + dot(nh, nb[...])) * invd + blo_ref[...]
    xhi = dot(tl, sc_[...]) + dot(th, sd[...]) + (dot(nl, ncc[...]) + dot(nh, nd[...])) * invd + bhi_ref[...]
    xlo = jnp.maximum(xlo, 0.0) + tl
    xhi = jnp.maximum(xhi, 0.0) + th
    hid = dot(xlo, w1a[...]) + dot(xhi, w1b[...]) + b1_ref[...]
    hid = jnp.maximum(hid, 0.0)
    out_ref[...] = dot(hid, w2_ref[...]) + b2_ref[...]


def _blk(minor):
    return pl.BlockSpec((BROW, minor), lambda i: (i, 0))


def _full(shape):
    return pl.BlockSpec(shape, lambda i: tuple(0 for _ in shape))


_PACKED = jax.ShapeDtypeStruct((NPQ, 128), jnp.float32)


def _embed_call(fpack, pbd, qbd, blo, bhi):
    return pl.pallas_call(
        _embed_body,
        grid=(GRID,),
        in_specs=[_blk(4 * IN_DIM), _full((4 * IN_DIM, 128)), _full((4 * IN_DIM, 128)),
                  _full((1, 128)), _full((1, 128))],
        out_specs=[_blk(128), _blk(128)],
        out_shape=[_PACKED, _PACKED],
    )(fpack, pbd, qbd, blo, bhi)


def _layer_call(residual, tl, th, nl, nh, dg, ws, blo, bhi):
    return pl.pallas_call(
        functools.partial(_layer_body, residual=residual),
        grid=(GRID,),
        in_specs=[_blk(128)] * 5 + [_full((128, 128))] * 8 + [_full((1, 128))] * 2,
        out_specs=[_blk(128), _blk(128)],
        out_shape=[_PACKED, _PACKED],
    )(tl, th, nl, nh, dg, *ws, blo, bhi)


def _layer_head_call(tl, th, nl, nh, dg, ws, blo, bhi, w1a, w1b, b1p, w2bd, b2p):
    return pl.pallas_call(
        _layer_head_body,
        grid=(GRID,),
        in_specs=[_blk(128)] * 5 + [_full((128, 128))] * 8 + [_full((1, 128))] * 2
                 + [_full((128, 128)), _full((128, 128)), _full((1, 128)),
                    _full((128, 4 * ODIM)), _full((1, 4 * ODIM))],
        out_specs=_blk(4 * ODIM),
        out_shape=jax.ShapeDtypeStruct((NPQ, 4 * ODIM), jnp.float32),
    )(tl, th, nl, nh, dg, *ws, blo, bhi, w1a, w1b, b1p, w2bd, b2p)


def kernel(features, edge_index, W_emb, b_emb, W_self, W_neigh, b_sage,
           bn_gamma, bn_beta, bn_mean, bn_var, W1, b1, W2, b2):
    ei = edge_index.astype(jnp.int32)
    srcp = ei[0]
    dstp = ei[1]

    scale = bn_gamma * lax.rsqrt(bn_var + 1e-5)
    bf = (b_sage - bn_mean) * scale + bn_beta
    Wsf = W_self * scale[:, :, None]
    Wnf = W_neigh * scale[:, :, None]
    eye4 = jnp.eye(4, dtype=jnp.float32)
    bd = lambda m: jnp.kron(eye4, m)
    layer_ws = []
    layer_bs = []
    for i in range(NLAYERS):
        ws = [bd(Wsf[i, :HALF, :HALF].T), bd(Wsf[i, :HALF, HALF:].T),
              bd(Wsf[i, HALF:, :HALF].T), bd(Wsf[i, HALF:, HALF:].T),
              bd(Wnf[i, :HALF, :HALF].T), bd(Wnf[i, :HALF, HALF:].T),
              bd(Wnf[i, HALF:, :HALF].T), bd(Wnf[i, HALF:, HALF:].T)]
        layer_ws.append(ws)
        layer_bs.append((jnp.tile(bf[i, :HALF], 4)[None, :],
                         jnp.tile(bf[i, HALF:], 4)[None, :]))
    pbd = bd(W_emb[:HALF, :].T)
    qbd = bd(W_emb[HALF:, :].T)
    eblo = jnp.tile(b_emb[:HALF], 4)[None, :]
    ebhi = jnp.tile(b_emb[HALF:], 4)[None, :]
    w1a = bd(W1[:, :HALF].T)
    w1b = bd(W1[:, HALF:].T)
    b1p = jnp.tile(b1, 4)[None, :]
    w2bd = bd(W2.T)
    b2p = jnp.tile(b2, 4)[None, :]

    fpack = jnp.pad(features.reshape(N // 4, 4 * IN_DIM), ((0, NPQ - N // 4), (0, 0)))

    agg0k, aggk = _sc_kernels()
    hl, hh = _embed_call(fpack, pbd, qbd, eblo, ebhi)
    degrep = None
    for i in range(NLAYERS):
        if i == 0:
            nl, nh, deg = agg0k(hl.reshape(NP, HALF), hh.reshape(NP, HALF),
                                srcp, dstp)
            degrep = jnp.repeat(deg, HALF).reshape(NPQ, 128)
        else:
            nl, nh = aggk(hl.reshape(NP, HALF), hh.reshape(NP, HALF),
                          srcp, dstp)
        nlp = nl.reshape(NPQ, 128)
        nhp = nh.reshape(NPQ, 128)
        if i < NLAYERS - 1:
            hl, hh = _layer_call(i > 0, hl, hh, nlp, nhp,
                                 degrep, layer_ws[i], *layer_bs[i])
        else:
            out = _layer_head_call(hl, hh, nlp, nhp, degrep,
                                   layer_ws[i], *layer_bs[i],
                                   w1a=w1a, w1b=w1b, b1p=b1p, w2bd=w2bd, b2p=b2p)
    return out.reshape(NP, ODIM)[:N]
